# baseline (speedup 1.0000x reference)
import functools

import jax
import jax.numpy as jnp
from jax import lax
from jax.experimental import pallas as pl
from jax.experimental.pallas import tpu as pltpu
from jax.experimental.pallas import tpu_sc as plsc

_B = 32
_D = 256
_H = 24
_W = 24
_HW = _H * _W

_NC = 2
_NS = 16
_ROWS = 2 * _D // _NS
_BPC = _B // _NC


def _tile_kernel(p_ref, row_ref, col_ref, pose_ref, tile_ref, pemb_ref):
    r = jax.lax.broadcasted_iota(jnp.int32, (_D, _HW), 0)
    l = jax.lax.broadcasted_iota(jnp.int32, (_D, _HW), 1)
    sel_col = (r == l % _W + 1).astype(jnp.float32)
    sel_row = (r == l // _W + 1).astype(jnp.float32)
    dn = (((0,), (0,)), ((), ()))
    hp = jax.lax.Precision.HIGHEST
    tile_ref[0:_D, :] = jax.lax.dot_general(
        col_ref[...], sel_col, dn, precision=hp,
        preferred_element_type=jnp.float32)
    tile_ref[_D:2 * _D, :] = jax.lax.dot_general(
        row_ref[...], sel_row, dn, precision=hp,
        preferred_element_type=jnp.float32)

    onehot = (jax.lax.broadcasted_iota(jnp.int32, (8, _D), 1)
              == p_ref[0]).astype(jnp.float32)
    pv = jax.lax.dot_general(onehot, pose_ref[...], (((1,), (0,)), ((), ())),
                             precision=hp,
                             preferred_element_type=jnp.float32)
    row = pv[0:1, :]
    pemb_ref[...] = jnp.broadcast_to(
        jnp.concatenate([row, row], axis=1), (_B, 2 * _D))


def _make_tile_and_pemb(p_arr, row_W, col_W, pose_W):
    return pl.pallas_call(
        _tile_kernel,
        in_specs=[
            pl.BlockSpec(memory_space=pltpu.SMEM),
            pl.BlockSpec(memory_space=pltpu.MemorySpace.VMEM),
            pl.BlockSpec(memory_space=pltpu.MemorySpace.VMEM),
            pl.BlockSpec(memory_space=pltpu.MemorySpace.VMEM),
        ],
        out_specs=[
            pl.BlockSpec(memory_space=pltpu.MemorySpace.VMEM),
            pl.BlockSpec(memory_space=pltpu.MemorySpace.VMEM),
        ],
        out_shape=[
            jax.ShapeDtypeStruct((2 * _D, _HW), jnp.float32),
            jax.ShapeDtypeStruct((_B, 2 * _D), jnp.float32),
        ],
    )(p_arr, row_W, col_W, pose_W)


@functools.partial(
    pl.kernel,
    mesh=plsc.VectorSubcoreMesh(core_axis_name="c", subcore_axis_name="s"),
    out_type=jax.ShapeDtypeStruct((_B * 2 * _D, _HW), jnp.float32),
    scratch_types=[
        pltpu.VMEM((_ROWS, _HW), jnp.float32),
        pltpu.SemaphoreType.DMA,
    ],
)
def _fanout_kernel(tile_hbm, m_hbm, chunk_vmem, sem):
    c = lax.axis_index("c")
    s = lax.axis_index("s")
    pltpu.sync_copy(tile_hbm.at[pl.ds(s * _ROWS, _ROWS)], chunk_vmem)
    copies = []
    for i in range(_BPC):
        b = c * _BPC + i
        dst = m_hbm.at[pl.ds(b * 2 * _D + s * _ROWS, _ROWS)]
        copies.append(pltpu.async_copy(chunk_vmem, dst, sem))
    for cp in copies:
        cp.wait()


def kernel(x, row_W, col_W, pose_W, p):
    b, c, h, w = x.shape
    p_arr = jnp.asarray(p, dtype=jnp.int32).reshape((1,))
    tile, p_emb = _make_tile_and_pemb(p_arr, row_W, col_W, pose_W)
    m_flat = _fanout_kernel(tile)
    return (p_emb, m_flat.reshape(b, 2 * _D, h, w))


# speedup vs baseline: 1.0011x; 1.0011x over previous
import functools

import jax
import jax.numpy as jnp
from jax import lax
from jax.experimental import pallas as pl
from jax.experimental.pallas import tpu as pltpu
from jax.experimental.pallas import tpu_sc as plsc

_B = 32
_D = 256
_H = 24
_W = 24
_HW = _H * _W

_NC = 2
_NS = 16
_ROWS = 2 * _D // _NS
_BPC = _B // _NC


def _tile_kernel(p_ref, row_ref, col_ref, pose_ref, tile_ref, pemb_ref):
    r = jax.lax.broadcasted_iota(jnp.int32, (_D, _HW), 0)
    l = jax.lax.broadcasted_iota(jnp.int32, (_D, _HW), 1)
    sel_col = (r == l % _W + 1).astype(jnp.float32)
    sel_row = (r == l // _W + 1).astype(jnp.float32)
    dn = (((0,), (0,)), ((), ()))
    hp = jax.lax.Precision.HIGHEST
    tile_ref[0:_D, :] = jax.lax.dot_general(
        col_ref[...], sel_col, dn, precision=hp,
        preferred_element_type=jnp.float32)
    tile_ref[_D:2 * _D, :] = jax.lax.dot_general(
        row_ref[...], sel_row, dn, precision=hp,
        preferred_element_type=jnp.float32)

    onehot = (jax.lax.broadcasted_iota(jnp.int32, (8, _D), 1)
              == p_ref[0]).astype(jnp.float32)
    pv = jax.lax.dot_general(onehot, pose_ref[...], (((1,), (0,)), ((), ())),
                             precision=hp,
                             preferred_element_type=jnp.float32)
    row = pv[0:1, :]
    pemb_ref[...] = jnp.broadcast_to(
        jnp.concatenate([row, row], axis=1), (_B, 2 * _D))


def _make_tile_and_pemb(p_arr, row_W, col_W, pose_W):
    return pl.pallas_call(
        _tile_kernel,
        in_specs=[
            pl.BlockSpec(memory_space=pltpu.SMEM),
            pl.BlockSpec(memory_space=pltpu.MemorySpace.VMEM),
            pl.BlockSpec(memory_space=pltpu.MemorySpace.VMEM),
            pl.BlockSpec(memory_space=pltpu.MemorySpace.VMEM),
        ],
        out_specs=[
            pl.BlockSpec(memory_space=pltpu.MemorySpace.VMEM),
            pl.BlockSpec(memory_space=pltpu.MemorySpace.VMEM),
        ],
        out_shape=[
            jax.ShapeDtypeStruct((2 * _D, _HW), jnp.float32),
            jax.ShapeDtypeStruct((_B, 2 * _D), jnp.float32),
        ],
    )(p_arr, row_W, col_W, pose_W)


@functools.partial(
    pl.kernel,
    mesh=plsc.VectorSubcoreMesh(core_axis_name="c", subcore_axis_name="s"),
    out_type=jax.ShapeDtypeStruct((_B * 2 * _D, _HW), jnp.float32),
    scratch_types=[
        pltpu.VMEM((_ROWS, _HW), jnp.float32),
        pltpu.SemaphoreType.DMA,
    ],
    compiler_params=pltpu.CompilerParams(use_tc_tiling_on_sc=True),
)
def _fanout_kernel(tile_hbm, m_hbm, chunk_vmem, sem):
    c = lax.axis_index("c")
    s = lax.axis_index("s")
    pltpu.sync_copy(tile_hbm.at[pl.ds(s * _ROWS, _ROWS)], chunk_vmem)
    copies = []
    for i in range(_BPC):
        b = c * _BPC + i
        dst = m_hbm.at[pl.ds(b * 2 * _D + s * _ROWS, _ROWS)]
        copies.append(pltpu.async_copy(chunk_vmem, dst, sem))
    for cp in copies:
        cp.wait()


def kernel(x, row_W, col_W, pose_W, p):
    b, c, h, w = x.shape
    p_arr = jnp.asarray(p, dtype=jnp.int32).reshape((1,))
    tile, p_emb = _make_tile_and_pemb(p_arr, row_W, col_W, pose_W)
    m_flat = _fanout_kernel(tile)
    return (p_emb, m_flat.reshape(b, 2 * _D, h, w))
